# Initial kernel scaffold; baseline (speedup 1.0000x reference)
import jax
import jax.numpy as jnp
from jax.experimental import pallas as pl


def kernel(embeds, labels):
    raise NotImplementedError("write your pallas kernel here")


# speedup vs baseline: 2.6774x; 2.6774x over previous
import jax
import jax.numpy as jnp
from jax.experimental import pallas as pl
from jax.experimental.pallas import tpu as pltpu

_N = 4096
_D = 512
_MARGIN = 0.5
_BLK = 256


def _triplet_kernel(e_blk_ref, e_all_ref, lab_col_ref, lab_row_ref, out_ref):
    i = pl.program_id(0)

    a = e_blk_ref[...]
    e = e_all_ref[...]
    n = e.shape[0]
    blk = a.shape[0]

    dot = jax.lax.dot_general(
        a, e, (((1,), (1,)), ((), ())), preferred_element_type=jnp.float32
    )

    sq_a = jnp.sum(a * a, axis=1, keepdims=True)
    sq_e = jnp.sum(e * e, axis=1, keepdims=True).T

    d2 = sq_a + sq_e - 2.0 * dot
    d2c = jnp.maximum(d2, 0.0)

    lab_a = lab_col_ref[...]
    lab_e = lab_row_ref[...]
    eq = lab_a == lab_e

    row_ids = jax.lax.broadcasted_iota(jnp.int32, (blk, n), 0) + i * blk
    col_ids = jax.lax.broadcasted_iota(jnp.int32, (blk, n), 1)
    self_m = row_ids == col_ids

    neg_inf = jnp.float32(-jnp.inf)
    pos_inf = jnp.float32(jnp.inf)
    pos_val = jnp.max(jnp.where(eq & (~self_m), d2c, neg_inf), axis=1, keepdims=True)
    neg_val = jnp.min(jnp.where(eq | self_m, pos_inf, d2c), axis=1, keepdims=True)

    d2c0 = d2c[:, 0:1]
    pos_val = jnp.where(pos_val == neg_inf, d2c0, pos_val)
    neg_val = jnp.where(neg_val == pos_inf, d2c0, neg_val)

    blk_loss = jnp.sum(
        jnp.maximum(pos_val - neg_val + _MARGIN, 0.0), keepdims=True
    ).reshape(1, 1)

    @pl.when(i == 0)
    def _init():
        out_ref[...] = jnp.zeros_like(out_ref)

    out_ref[...] += blk_loss


def kernel(embeds, labels):
    lab_col = labels.reshape(_N, 1)
    lab_row = labels.reshape(1, _N)

    total = pl.pallas_call(
        _triplet_kernel,
        grid=(_N // _BLK,),
        in_specs=[
            pl.BlockSpec((_BLK, _D), lambda i: (i, 0)),
            pl.BlockSpec((_N, _D), lambda i: (0, 0)),
            pl.BlockSpec((_BLK, 1), lambda i: (i, 0)),
            pl.BlockSpec((1, _N), lambda i: (0, 0)),
        ],
        out_specs=pl.BlockSpec((1, 1), lambda i: (0, 0)),
        out_shape=jax.ShapeDtypeStruct((1, 1), jnp.float32),
        compiler_params=pltpu.CompilerParams(
            dimension_semantics=("arbitrary",),
        ),
    )(embeds, embeds, lab_col, lab_row)

    return total[0, 0] / _N


# speedup vs baseline: 3.2593x; 1.2173x over previous
import jax
import jax.numpy as jnp
from jax.experimental import pallas as pl
from jax.experimental.pallas import tpu as pltpu

_N = 4096
_D = 512
_MARGIN = 0.5
_BLK = 256


def _triplet_kernel(e_blk_ref, e_all_ref, lab_col_ref, lab_row_ref, out_ref):
    i = pl.program_id(0)

    a = e_blk_ref[...]
    e = e_all_ref[...]
    n = e.shape[0]
    blk = a.shape[0]

    a2 = (-2.0 * a).astype(jnp.bfloat16)
    e16 = e.astype(jnp.bfloat16)
    dot = jax.lax.dot_general(
        a2, e16, (((1,), (1,)), ((), ())), preferred_element_type=jnp.float32
    )

    sq_a = jnp.sum(a * a, axis=1, keepdims=True)
    sq_e = jnp.sum(e * e, axis=1, keepdims=True).T

    m = sq_e + dot

    lab_a = lab_col_ref[...]
    lab_e = lab_row_ref[...]
    eq = lab_a == lab_e

    row_ids = jax.lax.broadcasted_iota(jnp.int32, (blk, n), 0) + i * blk
    col_ids = jax.lax.broadcasted_iota(jnp.int32, (blk, n), 1)
    self_m = row_ids == col_ids

    neg_inf = jnp.float32(-jnp.inf)
    pos_inf = jnp.float32(jnp.inf)
    pos_red = jnp.max(jnp.where(eq & (~self_m), m, neg_inf), axis=1, keepdims=True)
    neg_red = jnp.min(jnp.where(eq, pos_inf, m), axis=1, keepdims=True)

    m0 = m[:, 0:1]
    pos_red = jnp.where(pos_red == neg_inf, m0, pos_red)
    neg_red = jnp.where(neg_red == pos_inf, m0, neg_red)

    pos_val = jnp.maximum(sq_a + pos_red, 0.0)
    neg_val = jnp.maximum(sq_a + neg_red, 0.0)

    blk_loss = jnp.sum(
        jnp.maximum(pos_val - neg_val + _MARGIN, 0.0), keepdims=True
    ).reshape(1, 1)

    @pl.when(i == 0)
    def _init():
        out_ref[...] = jnp.zeros_like(out_ref)

    out_ref[...] += blk_loss


def kernel(embeds, labels):
    lab_col = labels.reshape(_N, 1)
    lab_row = labels.reshape(1, _N)

    total = pl.pallas_call(
        _triplet_kernel,
        grid=(_N // _BLK,),
        in_specs=[
            pl.BlockSpec((_BLK, _D), lambda i: (i, 0)),
            pl.BlockSpec((_N, _D), lambda i: (0, 0)),
            pl.BlockSpec((_BLK, 1), lambda i: (i, 0)),
            pl.BlockSpec((1, _N), lambda i: (0, 0)),
        ],
        out_specs=pl.BlockSpec((1, 1), lambda i: (0, 0)),
        out_shape=jax.ShapeDtypeStruct((1, 1), jnp.float32),
        compiler_params=pltpu.CompilerParams(
            dimension_semantics=("arbitrary",),
        ),
    )(embeds, embeds, lab_col, lab_row)

    return total[0, 0] / _N
